# baseline (speedup 1.0000x reference)
import functools

import jax
import jax.numpy as jnp
from jax import lax
from jax.experimental import pallas as pl
from jax.experimental.pallas import tpu as pltpu
from jax.experimental.pallas import tpu_sc as plsc

N = 10000
E = 160000
D = 256
H = 128
N_PAD = 10240
E_PAD = 163840
CHD = 128
CH = 128
NT = 16
RPT = N_PAD // NT
DUMMY_DST = N
NCH_W = E_PAD // 32 // CHD
NCH_T = E_PAD // NT // CH
IRING = 4
GRING = 2
DEG_GRP = 8

_mesh = plsc.VectorSubcoreMesh(core_axis_name="c", subcore_axis_name="s")


@functools.partial(
    pl.kernel,
    mesh=_mesh,
    out_type=jax.ShapeDtypeStruct((2, N_PAD), jnp.float32),
    scratch_types=[
        pltpu.VMEM((NCH_W, CHD), jnp.int32),
        pltpu.VMEM((CHD,), jnp.float32),
        pltpu.VMEM_SHARED((N_PAD,), jnp.float32),
        pltpu.SemaphoreType.DMA,
    ],
)
def _deg_kernel(dst2_hbm, zvec_hbm, degp_hbm, didx_v, ones_v, acc_s, sem):
    c = lax.axis_index("c")
    s = lax.axis_index("s")
    wid = s * 2 + c
    for j in range(CHD // 16):
        ones_v[pl.ds(j * 16, 16)] = jnp.ones((16,), jnp.float32)
    pltpu.sync_copy(dst2_hbm.at[pl.ds(wid * NCH_W, NCH_W)], didx_v)
    pltpu.sync_copy(zvec_hbm.at[pl.ds(s * RPT, RPT)],
                    acc_s.at[pl.ds(s * RPT, RPT)])
    plsc.subcore_barrier()

    def group(g, carry):
        for k in range(DEG_GRP):
            pltpu.async_copy(ones_v, acc_s.at[didx_v.at[g * DEG_GRP + k]],
                             sem, add=True)
        for k in range(DEG_GRP):
            pltpu.make_async_copy(
                ones_v, acc_s.at[didx_v.at[g * DEG_GRP + k]], sem).wait()
        return carry

    lax.fori_loop(0, NCH_W // DEG_GRP, group, 0)
    plsc.subcore_barrier()

    @pl.when(c == 0)
    def _():
        pltpu.sync_copy(acc_s.at[pl.ds(s * RPT, RPT)],
                        degp_hbm.at[0, pl.ds(s * RPT, RPT)])

    @pl.when(c == 1)
    def _():
        pltpu.sync_copy(acc_s.at[pl.ds(s * RPT, RPT)],
                        degp_hbm.at[1, pl.ds(s * RPT, RPT)])


CPC = (E_PAD // CH) // 2
NCH_T = CPC // NT


@functools.partial(
    pl.kernel,
    mesh=_mesh,
    compiler_params=pltpu.CompilerParams(use_tc_tiling_on_sc=False),
    out_type=(
        jax.ShapeDtypeStruct((N_PAD, 2, H), jnp.bfloat16),
        jax.ShapeDtypeStruct((N_PAD, 2, H), jnp.bfloat16),
    ),
    scratch_types=(
        [pltpu.VMEM((CH,), jnp.int32)] * IRING
        + [pltpu.VMEM((CH,), jnp.int32)] * IRING
        + [pltpu.VMEM((CH, 2, H), jnp.bfloat16)] * GRING
        + [pltpu.VMEM_SHARED((N_PAD, 2, H), jnp.bfloat16)]
        + [pltpu.SemaphoreType.DMA] * IRING
        + [pltpu.SemaphoreType.DMA] * GRING
    ),
)
def _scatter_kernel(h0_hbm, h1_hbm, src2_hbm, dst2_hbm, zmat_hbm,
                    a0_hbm, a1_hbm, *rest):
    sidx = rest[:IRING]
    didx = rest[IRING:2 * IRING]
    rows = rest[2 * IRING:2 * IRING + GRING]
    acc_s = rest[2 * IRING + GRING]
    isems = rest[2 * IRING + GRING + 1:2 * IRING + GRING + 1 + IRING]
    gsems = rest[2 * IRING + GRING + 1 + IRING:]
    c = lax.axis_index("c")
    s = lax.axis_index("s")
    cbase = c * CPC + s * NCH_T

    def fire_idx(i, q):
        pltpu.async_copy(src2_hbm.at[cbase + i], sidx[q], isems[q])
        pltpu.async_copy(dst2_hbm.at[cbase + i], didx[q], isems[q])

    def wait_idx(i, q):
        pltpu.make_async_copy(src2_hbm.at[cbase + i], sidx[q],
                              isems[q]).wait()
        pltpu.make_async_copy(dst2_hbm.at[cbase + i], didx[q],
                              isems[q]).wait()

    def fire_gather(q, r):
        @pl.when(c == 0)
        def _():
            pltpu.async_copy(h0_hbm.at[sidx[q]], rows[r], gsems[r])

        @pl.when(c == 1)
        def _():
            pltpu.async_copy(h1_hbm.at[sidx[q]], rows[r], gsems[r])

    for q in range(IRING):
        fire_idx(q, q)
    for r in range(GRING):
        wait_idx(r, r)
        fire_gather(r, r)
    pltpu.sync_copy(zmat_hbm.at[pl.ds(s * RPT, RPT)],
                    acc_s.at[pl.ds(s * RPT, RPT)])
    plsc.subcore_barrier()

    def group(g, carry):
        for r in range(IRING):
            i = g * IRING + r
            q = r
            rr = r % GRING
            pltpu.make_async_copy(h0_hbm.at[sidx[q]], rows[rr],
                                  gsems[rr]).wait()
            pltpu.sync_copy(rows[rr], acc_s.at[didx[q]], add=True)

            @pl.when(i + IRING < NCH_T)
            def _():
                fire_idx(i + IRING, q)

            @pl.when(i + GRING < NCH_T)
            def _():
                wait_idx(i + GRING, (r + GRING) % IRING)
                fire_gather((r + GRING) % IRING, rr)
        return carry

    lax.fori_loop(0, NCH_T // IRING, group, 0)
    plsc.subcore_barrier()

    @pl.when(c == 0)
    def _():
        pltpu.sync_copy(acc_s.at[pl.ds(s * RPT, RPT)],
                        a0_hbm.at[pl.ds(s * RPT, RPT)])

    @pl.when(c == 1)
    def _():
        pltpu.sync_copy(acc_s.at[pl.ds(s * RPT, RPT)],
                        a1_hbm.at[pl.ds(s * RPT, RPT)])


BLK = 1024


def _prep_body(x_ref, w_ref, dp_ref, h0_ref, h1_ref):
    y = jnp.dot(x_ref[...], w_ref[...],
                preferred_element_type=jnp.float32,
                precision=lax.Precision.HIGHEST)
    deg = jnp.maximum(dp_ref[0, :] + dp_ref[1, :], 1.0)
    norm = lax.rsqrt(deg)
    h = (y * norm[:, None]).astype(jnp.bfloat16).reshape(BLK, 2, H)
    h0_ref[...] = h
    h1_ref[...] = h


def _final_body(a0_ref, a1_ref, dp_ref, b_ref, out_ref):
    agg = (a0_ref[...].astype(jnp.float32)
           + a1_ref[...].astype(jnp.float32)).reshape(BLK, D)
    deg = jnp.maximum(dp_ref[0, :] + dp_ref[1, :], 1.0)
    norm = lax.rsqrt(deg)
    out_ref[...] = agg * norm[:, None] + b_ref[0, :][None, :]


_prep_call = pl.pallas_call(
    _prep_body,
    grid=(N_PAD // BLK,),
    in_specs=[
        pl.BlockSpec((BLK, D), lambda i: (i, 0)),
        pl.BlockSpec((D, D), lambda i: (0, 0)),
        pl.BlockSpec((2, BLK), lambda i: (0, i)),
    ],
    out_specs=[
        pl.BlockSpec((BLK, 2, H), lambda i: (i, 0, 0)),
        pl.BlockSpec((BLK, 2, H), lambda i: (i, 0, 0)),
    ],
    out_shape=[
        jax.ShapeDtypeStruct((N, 2, H), jnp.bfloat16),
        jax.ShapeDtypeStruct((N, 2, H), jnp.bfloat16),
    ],
)

_final_call = pl.pallas_call(
    _final_body,
    grid=(N_PAD // BLK,),
    in_specs=[
        pl.BlockSpec((BLK, 2, H), lambda i: (i, 0, 0)),
        pl.BlockSpec((BLK, 2, H), lambda i: (i, 0, 0)),
        pl.BlockSpec((2, BLK), lambda i: (0, i)),
        pl.BlockSpec((1, D), lambda i: (0, 0)),
    ],
    out_specs=pl.BlockSpec((BLK, D), lambda i: (i, 0)),
    out_shape=jax.ShapeDtypeStruct((N, D), jnp.float32),
)


def kernel(x, edge_index, W, b):
    src = edge_index[0]
    dst = edge_index[1]
    pad = E_PAD - E
    srcp = jnp.concatenate([src, jnp.zeros((pad,), jnp.int32)])
    dstp = jnp.concatenate([dst, jnp.full((pad,), DUMMY_DST, jnp.int32)])
    src2 = jnp.reshape(srcp, (E_PAD // CH, CH))
    dst2 = jnp.reshape(dstp, (E_PAD // CH, CH))
    dst2d = jnp.reshape(dstp, (E_PAD // CHD, CHD))
    zvec = jnp.zeros((N_PAD,), jnp.float32)
    zmat = jnp.zeros((N_PAD, 2, H), jnp.bfloat16)

    degp = _deg_kernel(dst2d, zvec)
    h0, h1 = _prep_call(x, W, degp)
    a0, a1 = _scatter_kernel(h0, h1, src2, dst2, zmat)
    out = _final_call(a0, a1, degp, jnp.reshape(b, (1, D)))
    return out


# speedup vs baseline: 1.3666x; 1.3666x over previous
import functools

import jax
import jax.numpy as jnp
from jax import lax
from jax.experimental import pallas as pl
from jax.experimental.pallas import tpu as pltpu
from jax.experimental.pallas import tpu_sc as plsc

N = 10000
E = 160000
D = 256
H = 128
N_PAD = 10240
E_PAD = 163840
CHD = 128
CH = 128
NT = 16
RPT = N_PAD // NT
DUMMY_DST = N
NCH_W = E_PAD // 32 // CHD
NCH_T = E_PAD // NT // CH
IRING = 4
GRING = 2
DEG_GRP = 8

_mesh = plsc.VectorSubcoreMesh(core_axis_name="c", subcore_axis_name="s")


@functools.partial(
    pl.kernel,
    mesh=_mesh,
    out_type=jax.ShapeDtypeStruct((2, N_PAD), jnp.float32),
    scratch_types=[
        pltpu.VMEM((NCH_W, CHD), jnp.int32),
        pltpu.VMEM((CHD,), jnp.float32),
        pltpu.VMEM_SHARED((N_PAD,), jnp.float32),
        pltpu.SemaphoreType.DMA,
    ],
)
def _deg_kernel(dst2_hbm, zvec_hbm, degp_hbm, didx_v, ones_v, acc_s, sem):
    c = lax.axis_index("c")
    s = lax.axis_index("s")
    wid = s * 2 + c
    for j in range(CHD // 16):
        ones_v[pl.ds(j * 16, 16)] = jnp.ones((16,), jnp.float32)
    pltpu.sync_copy(dst2_hbm.at[pl.ds(wid * NCH_W, NCH_W)], didx_v)
    pltpu.sync_copy(zvec_hbm.at[pl.ds(s * RPT, RPT)],
                    acc_s.at[pl.ds(s * RPT, RPT)])
    plsc.subcore_barrier()

    def group(g, carry):
        for k in range(DEG_GRP):
            pltpu.async_copy(ones_v, acc_s.at[didx_v.at[g * DEG_GRP + k]],
                             sem, add=True)
        for k in range(DEG_GRP):
            pltpu.make_async_copy(
                ones_v, acc_s.at[didx_v.at[g * DEG_GRP + k]], sem).wait()
        return carry

    lax.fori_loop(0, NCH_W // DEG_GRP, group, 0)
    plsc.subcore_barrier()

    @pl.when(c == 0)
    def _():
        pltpu.sync_copy(acc_s.at[pl.ds(s * RPT, RPT)],
                        degp_hbm.at[0, pl.ds(s * RPT, RPT)])

    @pl.when(c == 1)
    def _():
        pltpu.sync_copy(acc_s.at[pl.ds(s * RPT, RPT)],
                        degp_hbm.at[1, pl.ds(s * RPT, RPT)])


@functools.partial(
    pl.kernel,
    mesh=_mesh,
    out_type=(
        jax.ShapeDtypeStruct((N_PAD, H), jnp.float32),
        jax.ShapeDtypeStruct((N_PAD, H), jnp.float32),
    ),
    scratch_types=(
        [pltpu.VMEM((CH,), jnp.int32)] * IRING
        + [pltpu.VMEM((CH,), jnp.int32)] * IRING
        + [pltpu.VMEM((CH, H), jnp.float32)] * GRING
        + [pltpu.VMEM_SHARED((N_PAD, H), jnp.float32)]
        + [pltpu.SemaphoreType.DMA] * IRING
        + [pltpu.SemaphoreType.DMA] * GRING
    ),
)
def _scatter_kernel(hlo_hbm, hhi_hbm, src2_hbm, dst2_hbm, zmat_hbm,
                    alo_hbm, ahi_hbm, *rest):
    sidx = rest[:IRING]
    didx = rest[IRING:2 * IRING]
    rows = rest[2 * IRING:2 * IRING + GRING]
    acc_s = rest[2 * IRING + GRING]
    isems = rest[2 * IRING + GRING + 1:2 * IRING + GRING + 1 + IRING]
    gsems = rest[2 * IRING + GRING + 1 + IRING:]
    c = lax.axis_index("c")
    s = lax.axis_index("s")

    def fire_idx(i, q):
        pltpu.async_copy(src2_hbm.at[s * NCH_T + i], sidx[q], isems[q])
        pltpu.async_copy(dst2_hbm.at[s * NCH_T + i], didx[q], isems[q])

    def wait_idx(i, q):
        pltpu.make_async_copy(src2_hbm.at[s * NCH_T + i], sidx[q],
                              isems[q]).wait()
        pltpu.make_async_copy(dst2_hbm.at[s * NCH_T + i], didx[q],
                              isems[q]).wait()

    def fire_gather(q, r):
        @pl.when(c == 0)
        def _():
            pltpu.async_copy(hlo_hbm.at[sidx[q]], rows[r], gsems[r])

        @pl.when(c == 1)
        def _():
            pltpu.async_copy(hhi_hbm.at[sidx[q]], rows[r], gsems[r])

    for q in range(IRING):
        fire_idx(q, q)
    for r in range(GRING):
        wait_idx(r, r)
        fire_gather(r, r)
    pltpu.sync_copy(zmat_hbm.at[pl.ds(s * RPT, RPT)],
                    acc_s.at[pl.ds(s * RPT, RPT)])
    plsc.subcore_barrier()

    def group(g, carry):
        for r in range(IRING):
            i = g * IRING + r
            q = r
            rr = r % GRING
            pltpu.make_async_copy(hlo_hbm.at[sidx[q]], rows[rr],
                                  gsems[rr]).wait()
            pltpu.sync_copy(rows[rr], acc_s.at[didx[q]], add=True)

            @pl.when(i + IRING < NCH_T)
            def _():
                fire_idx(i + IRING, q)

            @pl.when(i + GRING < NCH_T)
            def _():
                wait_idx(i + GRING, (r + GRING) % IRING)
                fire_gather((r + GRING) % IRING, rr)
        return carry

    lax.fori_loop(0, NCH_T // IRING, group, 0)
    plsc.subcore_barrier()

    @pl.when(c == 0)
    def _():
        pltpu.sync_copy(acc_s.at[pl.ds(s * RPT, RPT)],
                        alo_hbm.at[pl.ds(s * RPT, RPT)])

    @pl.when(c == 1)
    def _():
        pltpu.sync_copy(acc_s.at[pl.ds(s * RPT, RPT)],
                        ahi_hbm.at[pl.ds(s * RPT, RPT)])


BLK = 1024


def _prep_body(x_ref, w_ref, dp_ref, hlo_ref, hhi_ref):
    y = jnp.dot(x_ref[...], w_ref[...],
                preferred_element_type=jnp.float32,
                precision=lax.Precision.HIGHEST)
    deg = jnp.maximum(dp_ref[0, :] + dp_ref[1, :], 1.0)
    norm = lax.rsqrt(deg)
    h = y * norm[:, None]
    hlo_ref[...] = h[:, :H]
    hhi_ref[...] = h[:, H:]


def _final_body(alo_ref, ahi_ref, dp_ref, b_ref, out_ref):
    agg = jnp.concatenate([alo_ref[...], ahi_ref[...]], axis=1)
    deg = jnp.maximum(dp_ref[0, :] + dp_ref[1, :], 1.0)
    norm = lax.rsqrt(deg)
    out_ref[...] = agg * norm[:, None] + b_ref[0, :][None, :]


_prep_call = pl.pallas_call(
    _prep_body,
    grid=(N_PAD // BLK,),
    in_specs=[
        pl.BlockSpec((BLK, D), lambda i: (i, 0)),
        pl.BlockSpec((D, D), lambda i: (0, 0)),
        pl.BlockSpec((2, BLK), lambda i: (0, i)),
    ],
    out_specs=[
        pl.BlockSpec((BLK, H), lambda i: (i, 0)),
        pl.BlockSpec((BLK, H), lambda i: (i, 0)),
    ],
    out_shape=[
        jax.ShapeDtypeStruct((N, H), jnp.float32),
        jax.ShapeDtypeStruct((N, H), jnp.float32),
    ],
)

_final_call = pl.pallas_call(
    _final_body,
    grid=(N_PAD // BLK,),
    in_specs=[
        pl.BlockSpec((BLK, H), lambda i: (i, 0)),
        pl.BlockSpec((BLK, H), lambda i: (i, 0)),
        pl.BlockSpec((2, BLK), lambda i: (0, i)),
        pl.BlockSpec((1, D), lambda i: (0, 0)),
    ],
    out_specs=pl.BlockSpec((BLK, D), lambda i: (i, 0)),
    out_shape=jax.ShapeDtypeStruct((N, D), jnp.float32),
)


def kernel(x, edge_index, W, b):
    src = edge_index[0]
    dst = edge_index[1]
    pad = E_PAD - E
    srcp = jnp.concatenate([src, jnp.zeros((pad,), jnp.int32)])
    dstp = jnp.concatenate([dst, jnp.full((pad,), DUMMY_DST, jnp.int32)])
    src2 = jnp.reshape(srcp, (E_PAD // CH, CH))
    dst2 = jnp.reshape(dstp, (E_PAD // CH, CH))
    dst2d = jnp.reshape(dstp, (E_PAD // CHD, CHD))
    zvec = jnp.zeros((N_PAD,), jnp.float32)
    zmat = jnp.zeros((N_PAD, H), jnp.float32)

    degp = _deg_kernel(dst2d, zvec)
    hlo, hhi = _prep_call(x, W, degp)
    alo, ahi = _scatter_kernel(hlo, hhi, src2, dst2, zmat)
    out = _final_call(alo, ahi, degp, jnp.reshape(b, (1, D)))
    return out


# speedup vs baseline: 1.3812x; 1.0107x over previous
import functools

import jax
import jax.numpy as jnp
from jax import lax
from jax.experimental import pallas as pl
from jax.experimental.pallas import tpu as pltpu
from jax.experimental.pallas import tpu_sc as plsc

N = 10000
E = 160000
D = 256
H = 128
N_PAD = 10240
E_PAD = 163840
CHD = 128
CH = 128
NT = 16
RPT = N_PAD // NT
DUMMY_DST = N
NCH_W = E_PAD // 32 // CHD
NCH_T = E_PAD // NT // CH
IRING = 4
GRING = 2
DEG_GRP = 8

_mesh = plsc.VectorSubcoreMesh(core_axis_name="c", subcore_axis_name="s")


@functools.partial(
    pl.kernel,
    mesh=_mesh,
    out_type=jax.ShapeDtypeStruct((2, N_PAD), jnp.float32),
    scratch_types=[
        pltpu.VMEM((NCH_W, CHD), jnp.int32),
        pltpu.VMEM((CHD,), jnp.float32),
        pltpu.VMEM_SHARED((N_PAD,), jnp.float32),
        pltpu.SemaphoreType.DMA,
    ],
)
def _deg_kernel(dst2_hbm, zvec_hbm, degp_hbm, didx_v, ones_v, acc_s, sem):
    c = lax.axis_index("c")
    s = lax.axis_index("s")
    wid = s * 2 + c
    for j in range(CHD // 16):
        ones_v[pl.ds(j * 16, 16)] = jnp.ones((16,), jnp.float32)
    pltpu.sync_copy(dst2_hbm.at[pl.ds(wid * NCH_W, NCH_W)], didx_v)
    pltpu.sync_copy(zvec_hbm.at[pl.ds(s * RPT, RPT)],
                    acc_s.at[pl.ds(s * RPT, RPT)])
    plsc.subcore_barrier()

    def group(g, carry):
        for k in range(DEG_GRP):
            pltpu.async_copy(ones_v, acc_s.at[didx_v.at[g * DEG_GRP + k]],
                             sem, add=True)
        for k in range(DEG_GRP):
            pltpu.make_async_copy(
                ones_v, acc_s.at[didx_v.at[g * DEG_GRP + k]], sem).wait()
        return carry

    lax.fori_loop(0, NCH_W // DEG_GRP, group, 0)
    plsc.subcore_barrier()

    @pl.when(c == 0)
    def _():
        pltpu.sync_copy(acc_s.at[pl.ds(s * RPT, RPT)],
                        degp_hbm.at[0, pl.ds(s * RPT, RPT)])

    @pl.when(c == 1)
    def _():
        pltpu.sync_copy(acc_s.at[pl.ds(s * RPT, RPT)],
                        degp_hbm.at[1, pl.ds(s * RPT, RPT)])


@functools.partial(
    pl.kernel,
    mesh=_mesh,
    out_type=(
        jax.ShapeDtypeStruct((N_PAD, H), jnp.float32),
        jax.ShapeDtypeStruct((N_PAD, H), jnp.float32),
    ),
    scratch_types=(
        [pltpu.VMEM((CH,), jnp.int32)] * IRING
        + [pltpu.VMEM((CH,), jnp.int32)] * IRING
        + [pltpu.VMEM((CH, H), jnp.float32)] * GRING
        + [pltpu.VMEM_SHARED((N_PAD, H), jnp.float32)]
        + [pltpu.SemaphoreType.DMA] * IRING
        + [pltpu.SemaphoreType.DMA] * GRING
    ),
)
def _scatter_kernel(hlo_hbm, hhi_hbm, src2_hbm, dst2_hbm, zmat_hbm,
                    alo_hbm, ahi_hbm, *rest):
    sidx = rest[:IRING]
    didx = rest[IRING:2 * IRING]
    rows = rest[2 * IRING:2 * IRING + GRING]
    acc_s = rest[2 * IRING + GRING]
    isems = rest[2 * IRING + GRING + 1:2 * IRING + GRING + 1 + IRING]
    gsems = rest[2 * IRING + GRING + 1 + IRING:]
    c = lax.axis_index("c")
    s = lax.axis_index("s")

    def fire_idx(i, q):
        pltpu.async_copy(src2_hbm.at[s * NCH_T + i], sidx[q], isems[q])
        pltpu.async_copy(dst2_hbm.at[s * NCH_T + i], didx[q], isems[q])

    def wait_idx(i, q):
        pltpu.make_async_copy(src2_hbm.at[s * NCH_T + i], sidx[q],
                              isems[q]).wait()
        pltpu.make_async_copy(dst2_hbm.at[s * NCH_T + i], didx[q],
                              isems[q]).wait()

    def fire_gather(q, r):
        @pl.when(c == 0)
        def _():
            pltpu.async_copy(hlo_hbm.at[sidx[q]], rows[r], gsems[r])

        @pl.when(c == 1)
        def _():
            pltpu.async_copy(hhi_hbm.at[sidx[q]], rows[r], gsems[r])

    for q in range(IRING):
        fire_idx(q, q)
    for r in range(GRING):
        wait_idx(r, r)
        fire_gather(r, r)
    pltpu.sync_copy(zmat_hbm.at[pl.ds(s * RPT, RPT)],
                    acc_s.at[pl.ds(s * RPT, RPT)])
    plsc.subcore_barrier()

    def group(g, carry):
        for r in range(IRING):
            i = g * IRING + r
            q = r
            rr = r % GRING
            pltpu.make_async_copy(hlo_hbm.at[sidx[q]], rows[rr],
                                  gsems[rr]).wait()
            pltpu.sync_copy(rows[rr], acc_s.at[didx[q]], add=True)

            @pl.when(i + IRING < NCH_T)
            def _():
                fire_idx(i + IRING, q)

            @pl.when(i + GRING < NCH_T)
            def _():
                wait_idx(i + GRING, (r + GRING) % IRING)
                fire_gather((r + GRING) % IRING, rr)
        return carry

    lax.fori_loop(0, NCH_T // IRING, group, 0)
    plsc.subcore_barrier()

    @pl.when(c == 0)
    def _():
        pltpu.sync_copy(acc_s.at[pl.ds(s * RPT, RPT)],
                        alo_hbm.at[pl.ds(s * RPT, RPT)])

    @pl.when(c == 1)
    def _():
        pltpu.sync_copy(acc_s.at[pl.ds(s * RPT, RPT)],
                        ahi_hbm.at[pl.ds(s * RPT, RPT)])


BLK = 1024


def _prep_body(x_ref, w_ref, dp_ref, hlo_ref, hhi_ref):
    y = jnp.dot(x_ref[...], w_ref[...],
                preferred_element_type=jnp.float32)
    deg = jnp.maximum(dp_ref[0, :] + dp_ref[1, :], 1.0)
    norm = lax.rsqrt(deg)
    h = y * norm[:, None]
    hlo_ref[...] = h[:, :H]
    hhi_ref[...] = h[:, H:]


def _final_body(alo_ref, ahi_ref, dp_ref, b_ref, out_ref):
    agg = jnp.concatenate([alo_ref[...], ahi_ref[...]], axis=1)
    deg = jnp.maximum(dp_ref[0, :] + dp_ref[1, :], 1.0)
    norm = lax.rsqrt(deg)
    out_ref[...] = agg * norm[:, None] + b_ref[0, :][None, :]


_prep_call = pl.pallas_call(
    _prep_body,
    grid=(N_PAD // BLK,),
    in_specs=[
        pl.BlockSpec((BLK, D), lambda i: (i, 0)),
        pl.BlockSpec((D, D), lambda i: (0, 0)),
        pl.BlockSpec((2, BLK), lambda i: (0, i)),
    ],
    out_specs=[
        pl.BlockSpec((BLK, H), lambda i: (i, 0)),
        pl.BlockSpec((BLK, H), lambda i: (i, 0)),
    ],
    out_shape=[
        jax.ShapeDtypeStruct((N, H), jnp.float32),
        jax.ShapeDtypeStruct((N, H), jnp.float32),
    ],
)

_final_call = pl.pallas_call(
    _final_body,
    grid=(N_PAD // BLK,),
    in_specs=[
        pl.BlockSpec((BLK, H), lambda i: (i, 0)),
        pl.BlockSpec((BLK, H), lambda i: (i, 0)),
        pl.BlockSpec((2, BLK), lambda i: (0, i)),
        pl.BlockSpec((1, D), lambda i: (0, 0)),
    ],
    out_specs=pl.BlockSpec((BLK, D), lambda i: (i, 0)),
    out_shape=jax.ShapeDtypeStruct((N, D), jnp.float32),
)


def kernel(x, edge_index, W, b):
    src = edge_index[0]
    dst = edge_index[1]
    pad = E_PAD - E
    srcp = jnp.concatenate([src, jnp.zeros((pad,), jnp.int32)])
    dstp = jnp.concatenate([dst, jnp.full((pad,), DUMMY_DST, jnp.int32)])
    src2 = jnp.reshape(srcp, (E_PAD // CH, CH))
    dst2 = jnp.reshape(dstp, (E_PAD // CH, CH))
    dst2d = jnp.reshape(dstp, (E_PAD // CHD, CHD))
    zvec = jnp.zeros((N_PAD,), jnp.float32)
    zmat = jnp.zeros((N_PAD, H), jnp.float32)

    degp = _deg_kernel(dst2d, zvec)
    hlo, hhi = _prep_call(x, W, degp)
    alo, ahi = _scatter_kernel(hlo, hhi, src2, dst2, zmat)
    out = _final_call(alo, ahi, degp, jnp.reshape(b, (1, D)))
    return out
